# baseline (speedup 1.0000x reference)
import functools

import jax
import jax.numpy as jnp
from jax import lax
from jax.experimental import pallas as pl
from jax.experimental.pallas import tpu as pltpu
from jax.experimental.pallas import tpu_sc as plsc

NC = 2
NS = 16
NW = NC * NS
LANES = 16


def _sc_detile_table(emb_t, tail_flat, V, D):
    TCOLS = V // 128
    per_w = TCOLS // NW
    CH = 4
    nch = per_w // CH
    CW = CH * 128
    vpw = per_w * 128
    E0 = NW * vpw
    extra_full = TCOLS - NW * per_w
    rem = V - TCOLS * 128
    assert nch >= 3 and nch % 2 == 1
    mesh = plsc.VectorSubcoreMesh(
        core_axis_name="c", subcore_axis_name="s",
        num_cores=NC, num_subcores=NS)

    @functools.partial(
        pl.kernel,
        out_type=jax.ShapeDtypeStruct((V * D,), jnp.float32),
        mesh=mesh,
        compiler_params=pltpu.CompilerParams(needs_layout_passes=False),
        scratch_types=[
            pltpu.VMEM((D, CW + 1), jnp.float32),
            pltpu.VMEM((D, CW + 1), jnp.float32),
            pltpu.VMEM((CW * D,), jnp.float32),
            pltpu.VMEM((CW * D,), jnp.float32),
            pltpu.SemaphoreType.DMA,
            pltpu.SemaphoreType.DMA,
            pltpu.SemaphoreType.DMA,
            pltpu.SemaphoreType.DMA,
        ],
    )
    def body(t_hbm, tail_hbm, out_hbm, ib0, ib1, ob0, ob1, si0, si1, so0, so1):
        wid = lax.axis_index("s") * NC + lax.axis_index("c")
        vb = pl.multiple_of(wid * vpw, 128)
        ibufs = (ib0, ib1)
        obufs = (ob0, ob1)
        sis = (si0, si1)
        sos = (so0, so1)
        d_lo = jnp.arange(LANES, dtype=jnp.int32)
        d_hi = d_lo + LANES

        def fire_in(c, b):
            start = pl.multiple_of(vb + c * CW, 128)
            pltpu.make_async_copy(
                t_hbm.at[:, pl.ds(start, CW)],
                ibufs[b].at[:, pl.ds(0, CW)], sis[b]).start()

        def drain_in(b):
            pltpu.make_async_copy(
                t_hbm.at[:, pl.ds(vb, CW)],
                ibufs[b].at[:, pl.ds(0, CW)], sis[b]).wait()

        def fire_out(c, b):
            pltpu.make_async_copy(
                obufs[b], out_hbm.at[pl.ds((vb + c * CW) * D, CW * D)],
                sos[b]).start()

        def drain_out(b):
            pltpu.make_async_copy(
                obufs[b], out_hbm.at[pl.ds(vb * D, CW * D)], sos[b]).wait()

        def transpose_chunk(b):
            ib = ibufs[b]
            ob = obufs[b]

            @plsc.parallel_loop(0, CW, step=8, unroll=2)
            def _(v8):
                vals = []
                for u in range(8):
                    vv = jnp.full((LANES,), 0, jnp.int32) + (v8 + u)
                    vals.append((plsc.load_gather(ib, [d_lo, vv]),
                                 plsc.load_gather(ib, [d_hi, vv])))
                for u, (lo, hi) in enumerate(vals):
                    ob[pl.ds((v8 + u) * D, LANES)] = lo
                    ob[pl.ds((v8 + u) * D + LANES, LANES)] = hi

        def process(c, b, first):
            drain_in(b)
            if not first:
                drain_out(b)
            transpose_chunk(b)
            fire_out(c, b)
            fire_in(jnp.minimum(c + 2, nch - 1), b)

        fire_in(0, 0)
        fire_in(1, 1)
        process(0, 0, True)
        process(1, 1, True)

        def pbody(p, _):
            process(2 * p, 0, False)
            process(2 * p + 1, 1, False)
            return 0

        lax.fori_loop(1, (nch - 1) // 2, pbody, 0)
        process(nch - 1, 0, False)
        drain_in(0)
        drain_in(1)
        drain_out(0)
        drain_out(1)

        @pl.when(wid < extra_full)
        def _():
            v0 = pl.multiple_of(E0 + wid * 128, 128)
            pltpu.sync_copy(t_hbm.at[:, pl.ds(v0, 128)],
                            ib0.at[:, pl.ds(0, 128)])

            def ebody(v, _):
                vv = jnp.full((LANES,), 0, jnp.int32) + v
                lo = plsc.load_gather(ib0, [d_lo, vv])
                hi = plsc.load_gather(ib0, [d_hi, vv])
                ob0[pl.ds(v * D, LANES)] = lo
                ob0[pl.ds(v * D + LANES, LANES)] = hi
                return 0

            lax.fori_loop(0, 128, ebody, 0)
            pltpu.sync_copy(ob0.at[pl.ds(0, 128 * D)],
                            out_hbm.at[pl.ds(v0 * D, 128 * D)])

        if rem:
            @pl.when(wid == extra_full)
            def _():
                v0 = E0 + extra_full * 128
                pltpu.sync_copy(tail_hbm, ob0.at[pl.ds(0, rem * D)])
                pltpu.sync_copy(ob0.at[pl.ds(0, rem * D)],
                                out_hbm.at[pl.ds(v0 * D, rem * D)])

    return body(emb_t, tail_flat)


def _sc_sum_pool(x_r, emb_table, B, H, D, spw, chunk):
    h2 = H // 2
    nchunks = spw // chunk
    mesh = plsc.VectorSubcoreMesh(
        core_axis_name="c", subcore_axis_name="s",
        num_cores=NC, num_subcores=NS)

    @functools.partial(
        pl.kernel,
        out_type=jax.ShapeDtypeStruct((B, D), jnp.float32),
        mesh=mesh,
        compiler_params=pltpu.CompilerParams(use_tc_tiling_on_sc=False),
        scratch_types=[
            pltpu.VMEM((chunk, 2, h2), jnp.int32),
            pltpu.VMEM((chunk, 2, h2), jnp.int32),
            pltpu.VMEM((2, h2, D), jnp.float32),
            pltpu.VMEM((2, h2, D), jnp.float32),
            pltpu.VMEM((spw, D), jnp.float32),
            pltpu.SemaphoreType.DMA,
            pltpu.SemaphoreType.DMA,
            pltpu.SemaphoreType.DMA,
        ],
    )
    def body(x_hbm, emb_hbm, out_hbm, ibuf0, ibuf1, rbuf0, rbuf1,
             pooled, semi, sem0, sem1):
        wid = lax.axis_index("s") * NC + lax.axis_index("c")
        base = wid * spw
        ibufs = (ibuf0, ibuf1)
        rbufs = (rbuf0, rbuf1)
        sems = (sem0, sem1)

        def idx_copy(c):
            pltpu.make_async_copy(
                x_hbm.at[pl.ds(base + c * chunk, chunk)],
                ibufs[c % 2], semi).start()

        def idx_wait(c):
            pltpu.make_async_copy(
                x_hbm.at[pl.ds(base + c * chunk, chunk)],
                ibufs[c % 2], semi).wait()

        def fire(ib, i, rbi):
            for j in range(2):
                pltpu.make_async_copy(
                    emb_hbm.at[ibufs[ib].at[i, j]],
                    rbufs[rbi].at[j], sems[rbi]).start()

        def drain(ib, i, rbi):
            for j in range(2):
                pltpu.make_async_copy(
                    emb_hbm.at[ibufs[ib].at[i, j]],
                    rbufs[rbi].at[j], sems[rbi]).wait()

        def reduce(rbi, sl):
            rb = rbufs[rbi]
            zero = jnp.zeros((LANES,), jnp.float32)
            LO = pl.ds(0, LANES)
            HI = pl.ds(LANES, LANES)

            def rbody(r4, accs):
                a0, a1, a2, a3, a4, a5, a6, a7 = accs
                r = 4 * r4
                a0 = a0 + (rb[0, r, LO] + rb[0, r + 2, LO])
                a1 = a1 + (rb[0, r, HI] + rb[0, r + 2, HI])
                a2 = a2 + (rb[1, r, LO] + rb[1, r + 2, LO])
                a3 = a3 + (rb[1, r, HI] + rb[1, r + 2, HI])
                a4 = a4 + (rb[0, r + 1, LO] + rb[0, r + 3, LO])
                a5 = a5 + (rb[0, r + 1, HI] + rb[0, r + 3, HI])
                a6 = a6 + (rb[1, r + 1, LO] + rb[1, r + 3, LO])
                a7 = a7 + (rb[1, r + 1, HI] + rb[1, r + 3, HI])
                return (a0, a1, a2, a3, a4, a5, a6, a7)

            a = lax.fori_loop(0, h2 // 4, rbody, (zero,) * 8)
            lo = (a[0] + a[2]) + (a[4] + a[6])
            hi = (a[1] + a[3]) + (a[5] + a[7])
            pooled[sl, LO] = lo
            pooled[sl, HI] = hi

        idx_copy(0)
        idx_wait(0)
        for c in range(nchunks):
            ib = c % 2
            if c + 1 < nchunks:
                idx_copy(c + 1)
            fire(ib, 0, 0)
            fire(ib, 1, 1)

            def pbody(p, _, ib=ib, c=c):
                i0 = 2 * p
                last = chunk - 1
                drain(ib, i0, 0)
                reduce(0, c * chunk + i0)
                fire(ib, jnp.minimum(i0 + 2, last), 0)
                drain(ib, i0 + 1, 1)
                reduce(1, c * chunk + i0 + 1)
                fire(ib, jnp.minimum(i0 + 3, last), 1)
                return 0

            lax.fori_loop(0, chunk // 2, pbody, 0)
            drain(ib, chunk - 1, 0)
            drain(ib, chunk - 1, 1)
            if c + 1 < nchunks:
                idx_wait(c + 1)

        pltpu.sync_copy(pooled, out_hbm.at[pl.ds(base, spw)])

    return body(x_r, emb_table)


def _tc_linear_relu(sums, fc_w, fc_b2, inv_h, B, D):
    nblk = 8
    blk = B // nblk

    def body(s_ref, w_ref, b_ref, o_ref):
        pooled = s_ref[...] * inv_h
        acc = lax.dot_general(
            pooled, w_ref[...], (((1,), (1,)), ((), ())),
            preferred_element_type=jnp.float32)
        o_ref[...] = jnp.maximum(acc + b_ref[...], 0.0)

    return pl.pallas_call(
        body,
        out_shape=jax.ShapeDtypeStruct((B, D), jnp.float32),
        grid=(nblk,),
        in_specs=[
            pl.BlockSpec((blk, D), lambda i: (i, 0)),
            pl.BlockSpec((D, D), lambda i: (0, 0)),
            pl.BlockSpec((1, D), lambda i: (0, 0)),
        ],
        out_specs=pl.BlockSpec((blk, D), lambda i: (i, 0)),
    )(sums, fc_w, fc_b2)


def kernel(x, emb_table, fc_w, fc_b):
    B, H = x.shape
    D = emb_table.shape[1]
    assert B % NW == 0 and H % 2 == 0 and H // 2 <= 128 and D == 2 * LANES
    spw = B // NW
    chunk = 128
    assert spw % chunk == 0 and chunk % 2 == 0

    x_r = x.astype(jnp.int32).reshape(B, 2, H // 2)
    V = emb_table.shape[0]
    n_tail = V % 128
    tail_flat = emb_table[V - n_tail:].reshape(-1)
    lin_flat = _sc_detile_table(emb_table.T, tail_flat, V, D)
    lin_table = lin_flat.reshape(V, D)
    sums = _sc_sum_pool(x_r, lin_table, B, H, D, spw, chunk)
    return _tc_linear_relu(sums, fc_w, fc_b.reshape(1, D), 1.0 / H, B, D)


# speedup vs baseline: 1.7009x; 1.7009x over previous
import functools

import jax
import jax.numpy as jnp
from jax import lax
from jax.experimental import pallas as pl
from jax.experimental.pallas import tpu as pltpu
from jax.experimental.pallas import tpu_sc as plsc

NC = 2
NS = 16
NW = NC * NS
LANES = 16


def _sc_detile_table(emb_t, tail_flat, V, D):
    D2 = D // 2
    RW = D2 + 1
    TCOLS = V // 128
    per_w = TCOLS // NW
    CH = 4
    nch = per_w // CH
    CW = CH * 128
    vpw = per_w * 128
    E0 = NW * vpw
    extra_full = TCOLS - NW * per_w
    rem = V - TCOLS * 128
    assert nch >= 3 and nch % 2 == 1
    mesh = plsc.VectorSubcoreMesh(
        core_axis_name="c", subcore_axis_name="s",
        num_cores=NC, num_subcores=NS)

    @functools.partial(
        pl.kernel,
        out_type=jax.ShapeDtypeStruct((V * D2,), jnp.int32),
        mesh=mesh,
        compiler_params=pltpu.CompilerParams(needs_layout_passes=False),
        scratch_types=[
            pltpu.VMEM((D, CW), jnp.float32),
            pltpu.VMEM((D, CW), jnp.float32),
            pltpu.VMEM((CW * RW,), jnp.int32),
            pltpu.VMEM((CW * D2,), jnp.int32),
            pltpu.VMEM((CW * D2,), jnp.int32),
            pltpu.SemaphoreType.DMA,
            pltpu.SemaphoreType.DMA,
            pltpu.SemaphoreType.DMA,
            pltpu.SemaphoreType.DMA,
        ],
    )
    def body(t_hbm, tail_hbm, out_hbm, ib0, ib1, pb, ob0, ob1,
             si0, si1, so0, so1):
        wid = lax.axis_index("s") * NC + lax.axis_index("c")
        vb = pl.multiple_of(wid * vpw, 128)
        ibufs = (ib0, ib1)
        obufs = (ob0, ob1)
        sis = (si0, si1)
        sos = (so0, so1)
        iota16 = jnp.arange(LANES, dtype=jnp.int32)
        iota_rw = iota16 * RW

        def fire_in(c, b):
            start = pl.multiple_of(vb + c * CW, 128)
            pltpu.make_async_copy(
                t_hbm.at[:, pl.ds(start, CW)], ibufs[b], sis[b]).start()

        def drain_in(b):
            pltpu.make_async_copy(
                t_hbm.at[:, pl.ds(vb, CW)], ibufs[b], sis[b]).wait()

        def fire_out(c, b):
            pltpu.make_async_copy(
                obufs[b], out_hbm.at[pl.ds((vb + c * CW) * D2, CW * D2)],
                sos[b]).start()

        def drain_out(b):
            pltpu.make_async_copy(
                obufs[b], out_hbm.at[pl.ds(vb * D2, CW * D2)], sos[b]).wait()

        def transpose_block(ib, ob, nrows):
            for d in range(D2):
                @plsc.parallel_loop(0, nrows, step=LANES)
                def _(v0, d=d):
                    a = ib[d, pl.ds(v0, LANES)]
                    bvals = ib[d + D2, pl.ds(v0, LANES)]
                    w = plsc.bitcast(
                        plsc.pack(a, bvals, format=plsc.PackFormat.INTERLEAVED),
                        jnp.int32)
                    plsc.store_scatter(pb, [iota_rw + (v0 * RW + d)], w)
            @plsc.parallel_loop(0, nrows, step=2, unroll=2)
            def _(r):
                ob[pl.ds(r * D2, LANES)] = pb[pl.ds(r * RW, LANES)]
                ob[pl.ds((r + 1) * D2, LANES)] = pb[pl.ds((r + 1) * RW, LANES)]

        def process(c, b, first):
            drain_in(b)
            if not first:
                drain_out(b)
            transpose_block(ibufs[b], obufs[b], CW)
            fire_out(c, b)
            fire_in(jnp.minimum(c + 2, nch - 1), b)

        fire_in(0, 0)
        fire_in(1, 1)
        process(0, 0, True)
        process(1, 1, True)

        def pbody(p, _):
            process(2 * p, 0, False)
            process(2 * p + 1, 1, False)
            return 0

        lax.fori_loop(1, (nch - 1) // 2, pbody, 0)
        process(nch - 1, 0, False)
        drain_in(0)
        drain_in(1)
        drain_out(0)
        drain_out(1)

        @pl.when(wid < extra_full)
        def _():
            v0 = pl.multiple_of(E0 + wid * 128, 128)
            pltpu.sync_copy(t_hbm.at[:, pl.ds(v0, 128)],
                            ib0.at[:, pl.ds(0, 128)])
            transpose_block(ib0, ob0, 128)
            pltpu.sync_copy(ob0.at[pl.ds(0, 128 * D2)],
                            out_hbm.at[pl.ds(v0 * D2, 128 * D2)])

        if rem:
            @pl.when(wid == extra_full)
            def _():
                v0 = E0 + extra_full * 128
                pltpu.sync_copy(tail_hbm, ob0.at[pl.ds(0, rem * D2)])
                pltpu.sync_copy(ob0.at[pl.ds(0, rem * D2)],
                                out_hbm.at[pl.ds(v0 * D2, rem * D2)])

    return body(emb_t, tail_flat)


def _sc_sum_pool(x_r, emb_table, B, H, D, spw, chunk):
    h2 = H // 2
    nchunks = spw // chunk
    mesh = plsc.VectorSubcoreMesh(
        core_axis_name="c", subcore_axis_name="s",
        num_cores=NC, num_subcores=NS)

    @functools.partial(
        pl.kernel,
        out_type=jax.ShapeDtypeStruct((B, D), jnp.float32),
        mesh=mesh,
        compiler_params=pltpu.CompilerParams(
            use_tc_tiling_on_sc=False, needs_layout_passes=False),
        scratch_types=[
            pltpu.VMEM((chunk, 2, h2), jnp.int32),
            pltpu.VMEM((chunk, 2, h2), jnp.int32),
            pltpu.VMEM((2, h2, D // 2), jnp.int32),
            pltpu.VMEM((2, h2, D // 2), jnp.int32),
            pltpu.VMEM((spw, D), jnp.float32),
            pltpu.SemaphoreType.DMA,
            pltpu.SemaphoreType.DMA,
            pltpu.SemaphoreType.DMA,
        ],
    )
    def body(x_hbm, emb_hbm, out_hbm, ibuf0, ibuf1, rbuf0, rbuf1,
             pooled, semi, sem0, sem1):
        wid = lax.axis_index("s") * NC + lax.axis_index("c")
        base = wid * spw
        ibufs = (ibuf0, ibuf1)
        rbufs = (rbuf0, rbuf1)
        sems = (sem0, sem1)

        def idx_copy(c):
            pltpu.make_async_copy(
                x_hbm.at[pl.ds(base + c * chunk, chunk)],
                ibufs[c % 2], semi).start()

        def idx_wait(c):
            pltpu.make_async_copy(
                x_hbm.at[pl.ds(base + c * chunk, chunk)],
                ibufs[c % 2], semi).wait()

        def fire(ib, i, rbi):
            for j in range(2):
                pltpu.make_async_copy(
                    emb_hbm.at[ibufs[ib].at[i, j]],
                    rbufs[rbi].at[j], sems[rbi]).start()

        def drain(ib, i, rbi):
            for j in range(2):
                pltpu.make_async_copy(
                    emb_hbm.at[ibufs[ib].at[i, j]],
                    rbufs[rbi].at[j], sems[rbi]).wait()

        def reduce(rbi, sl):
            rb = rbufs[rbi]
            zero = jnp.zeros((LANES,), jnp.float32)
            LO = pl.ds(0, LANES)
            HI = pl.ds(LANES, LANES)
            mask_hi = jnp.full((LANES,), -65536, jnp.int32)

            def lo_half(w):
                return plsc.bitcast(w << 16, jnp.float32)

            def hi_half(w):
                return plsc.bitcast(w & mask_hi, jnp.float32)

            def rbody(r4, accs):
                a0, a1, a2, a3, a4, a5, a6, a7 = accs
                r = 4 * r4
                w00 = rb[0, r, :]
                w01 = rb[0, r + 1, :]
                w02 = rb[0, r + 2, :]
                w03 = rb[0, r + 3, :]
                w10 = rb[1, r, :]
                w11 = rb[1, r + 1, :]
                w12 = rb[1, r + 2, :]
                w13 = rb[1, r + 3, :]
                a0 = a0 + (lo_half(w00) + lo_half(w02))
                a1 = a1 + (hi_half(w00) + hi_half(w02))
                a2 = a2 + (lo_half(w10) + lo_half(w12))
                a3 = a3 + (hi_half(w10) + hi_half(w12))
                a4 = a4 + (lo_half(w01) + lo_half(w03))
                a5 = a5 + (hi_half(w01) + hi_half(w03))
                a6 = a6 + (lo_half(w11) + lo_half(w13))
                a7 = a7 + (hi_half(w11) + hi_half(w13))
                return (a0, a1, a2, a3, a4, a5, a6, a7)

            a = lax.fori_loop(0, h2 // 4, rbody, (zero,) * 8)
            lo = (a[0] + a[2]) + (a[4] + a[6])
            hi = (a[1] + a[3]) + (a[5] + a[7])
            pooled[sl, LO] = lo
            pooled[sl, HI] = hi

        idx_copy(0)
        idx_wait(0)
        for c in range(nchunks):
            ib = c % 2
            if c + 1 < nchunks:
                idx_copy(c + 1)
            fire(ib, 0, 0)
            fire(ib, 1, 1)

            def pbody(p, _, ib=ib, c=c):
                i0 = 2 * p
                last = chunk - 1
                drain(ib, i0, 0)
                reduce(0, c * chunk + i0)
                fire(ib, jnp.minimum(i0 + 2, last), 0)
                drain(ib, i0 + 1, 1)
                reduce(1, c * chunk + i0 + 1)
                fire(ib, jnp.minimum(i0 + 3, last), 1)
                return 0

            lax.fori_loop(0, chunk // 2, pbody, 0)
            drain(ib, chunk - 1, 0)
            drain(ib, chunk - 1, 1)
            if c + 1 < nchunks:
                idx_wait(c + 1)

        pltpu.sync_copy(pooled, out_hbm.at[pl.ds(base, spw)])

    return body(x_r, emb_table)


def _tc_linear_relu(sums, fc_w, fc_b2, inv_h, B, D):
    nblk = 8
    blk = B // nblk

    def body(s_ref, w_ref, b_ref, o_ref):
        pooled = s_ref[...] * inv_h
        acc = lax.dot_general(
            pooled, w_ref[...], (((1,), (1,)), ((), ())),
            preferred_element_type=jnp.float32)
        o_ref[...] = jnp.maximum(acc + b_ref[...], 0.0)

    return pl.pallas_call(
        body,
        out_shape=jax.ShapeDtypeStruct((B, D), jnp.float32),
        grid=(nblk,),
        in_specs=[
            pl.BlockSpec((blk, D), lambda i: (i, 0)),
            pl.BlockSpec((D, D), lambda i: (0, 0)),
            pl.BlockSpec((1, D), lambda i: (0, 0)),
        ],
        out_specs=pl.BlockSpec((blk, D), lambda i: (i, 0)),
    )(sums, fc_w, fc_b2)


def kernel(x, emb_table, fc_w, fc_b):
    B, H = x.shape
    D = emb_table.shape[1]
    assert B % NW == 0 and H % 2 == 0 and H // 2 <= 128 and D == 2 * LANES
    spw = B // NW
    chunk = 128
    assert spw % chunk == 0 and chunk % 2 == 0

    x_r = x.astype(jnp.int32).reshape(B, 2, H // 2)
    V = emb_table.shape[0]
    n_tail = V % 128
    tail = emb_table[V - n_tail:]
    tail_bits = jax.lax.bitcast_convert_type(
        tail.astype(jnp.bfloat16), jnp.uint16).astype(jnp.int32)
    tail_packed = (tail_bits[:, : D // 2]
                   | (tail_bits[:, D // 2:] << 16)).reshape(-1)
    lin_flat = _sc_detile_table(emb_table.T, tail_packed, V, D)
    lin_table = lin_flat.reshape(V, D // 2)
    sums = _sc_sum_pool(x_r, lin_table, B, H, D, spw, chunk)
    return _tc_linear_relu(sums, fc_w, fc_b.reshape(1, D), 1.0 / H, B, D)


# speedup vs baseline: 2.2088x; 1.2985x over previous
import functools

import jax
import jax.numpy as jnp
from jax import lax
from jax.experimental import pallas as pl
from jax.experimental.pallas import tpu as pltpu
from jax.experimental.pallas import tpu_sc as plsc

NC = 2
NS = 16
NW = NC * NS
LANES = 16


def _sc_detile_table(emb_t, tail_flat, V, D):
    D2 = D // 2
    RW = D2 + 1
    TCOLS = V // 128
    per_w = TCOLS // NW
    CH = 4
    nch = per_w // CH
    CW = CH * 128
    vpw = per_w * 128
    E0 = NW * vpw
    extra_full = TCOLS - NW * per_w
    rem = V - TCOLS * 128
    assert nch >= 3 and nch % 2 == 1
    mesh = plsc.VectorSubcoreMesh(
        core_axis_name="c", subcore_axis_name="s",
        num_cores=NC, num_subcores=NS)

    @functools.partial(
        pl.kernel,
        out_type=jax.ShapeDtypeStruct((V * D2,), jnp.int32),
        mesh=mesh,
        compiler_params=pltpu.CompilerParams(needs_layout_passes=False),
        scratch_types=[
            pltpu.VMEM((D, CW), jnp.float32),
            pltpu.VMEM((D, CW), jnp.float32),
            pltpu.VMEM((CW * RW,), jnp.int32),
            pltpu.VMEM((CW * D2,), jnp.int32),
            pltpu.VMEM((CW * D2,), jnp.int32),
            pltpu.SemaphoreType.DMA,
            pltpu.SemaphoreType.DMA,
            pltpu.SemaphoreType.DMA,
            pltpu.SemaphoreType.DMA,
        ],
    )
    def body(t_hbm, tail_hbm, out_hbm, ib0, ib1, pb, ob0, ob1,
             si0, si1, so0, so1):
        wid = lax.axis_index("s") * NC + lax.axis_index("c")
        vb = pl.multiple_of(wid * vpw, 128)
        ibufs = (ib0, ib1)
        obufs = (ob0, ob1)
        sis = (si0, si1)
        sos = (so0, so1)
        iota16 = jnp.arange(LANES, dtype=jnp.int32)
        iota_rw = iota16 * RW

        def fire_in(c, b):
            start = pl.multiple_of(vb + c * CW, 128)
            pltpu.make_async_copy(
                t_hbm.at[:, pl.ds(start, CW)], ibufs[b], sis[b]).start()

        def drain_in(b):
            pltpu.make_async_copy(
                t_hbm.at[:, pl.ds(vb, CW)], ibufs[b], sis[b]).wait()

        def fire_out(c, b):
            pltpu.make_async_copy(
                obufs[b], out_hbm.at[pl.ds((vb + c * CW) * D2, CW * D2)],
                sos[b]).start()

        def drain_out(b):
            pltpu.make_async_copy(
                obufs[b], out_hbm.at[pl.ds(vb * D2, CW * D2)], sos[b]).wait()

        def transpose_block(ib, ob, nrows):
            for d in range(D2):
                @plsc.parallel_loop(0, nrows, step=LANES)
                def _(v0, d=d):
                    a = ib[d, pl.ds(v0, LANES)]
                    bvals = ib[d + D2, pl.ds(v0, LANES)]
                    w = plsc.bitcast(
                        plsc.pack(a, bvals, format=plsc.PackFormat.INTERLEAVED),
                        jnp.int32)
                    plsc.store_scatter(pb, [iota_rw + (v0 * RW + d)], w)
            @plsc.parallel_loop(0, nrows, step=2, unroll=2)
            def _(r):
                ob[pl.ds(r * D2, LANES)] = pb[pl.ds(r * RW, LANES)]
                ob[pl.ds((r + 1) * D2, LANES)] = pb[pl.ds((r + 1) * RW, LANES)]

        def process(c, b, first):
            drain_in(b)
            if not first:
                drain_out(b)
            transpose_block(ibufs[b], obufs[b], CW)
            fire_out(c, b)
            fire_in(jnp.minimum(c + 2, nch - 1), b)

        fire_in(0, 0)
        fire_in(1, 1)
        process(0, 0, True)
        process(1, 1, True)

        def pbody(p, _):
            process(2 * p, 0, False)
            process(2 * p + 1, 1, False)
            return 0

        lax.fori_loop(1, (nch - 1) // 2, pbody, 0)
        process(nch - 1, 0, False)
        drain_in(0)
        drain_in(1)
        drain_out(0)
        drain_out(1)

        @pl.when(wid < extra_full)
        def _():
            v0 = pl.multiple_of(E0 + wid * 128, 128)
            pltpu.sync_copy(t_hbm.at[:, pl.ds(v0, 128)],
                            ib0.at[:, pl.ds(0, 128)])
            transpose_block(ib0, ob0, 128)
            pltpu.sync_copy(ob0.at[pl.ds(0, 128 * D2)],
                            out_hbm.at[pl.ds(v0 * D2, 128 * D2)])

        if rem:
            @pl.when(wid == extra_full)
            def _():
                v0 = E0 + extra_full * 128
                pltpu.sync_copy(tail_hbm, ob0.at[pl.ds(0, rem * D2)])
                pltpu.sync_copy(ob0.at[pl.ds(0, rem * D2)],
                                out_hbm.at[pl.ds(v0 * D2, rem * D2)])

    return body(emb_t, tail_flat)


def _sc_sum_pool(x_r, emb_table, B, H, D, spw, chunk):
    h2 = H // 2
    nchunks = spw // chunk
    mesh = plsc.VectorSubcoreMesh(
        core_axis_name="c", subcore_axis_name="s",
        num_cores=NC, num_subcores=NS)

    @functools.partial(
        pl.kernel,
        out_type=jax.ShapeDtypeStruct((B, D), jnp.float32),
        mesh=mesh,
        compiler_params=pltpu.CompilerParams(
            use_tc_tiling_on_sc=False, needs_layout_passes=False),
        scratch_types=[
            pltpu.VMEM((chunk, 2, h2), jnp.int32),
            pltpu.VMEM((chunk, 2, h2), jnp.int32),
            pltpu.VMEM((2, h2, D // 2), jnp.int32),
            pltpu.VMEM((2, h2, D // 2), jnp.int32),
            pltpu.VMEM((2, h2, D // 2), jnp.int32),
            pltpu.VMEM((2, h2, D // 2), jnp.int32),
            pltpu.VMEM((spw, D), jnp.float32),
            pltpu.SemaphoreType.DMA,
            pltpu.SemaphoreType.DMA,
            pltpu.SemaphoreType.DMA,
            pltpu.SemaphoreType.DMA,
            pltpu.SemaphoreType.DMA,
        ],
    )
    def body(x_hbm, emb_hbm, out_hbm, ibuf0, ibuf1, rbuf0, rbuf1, rbuf2, rbuf3,
             pooled, semi, sem0, sem1, sem2, sem3):
        wid = lax.axis_index("s") * NC + lax.axis_index("c")
        base = wid * spw
        ibufs = (ibuf0, ibuf1)
        rbufs = (rbuf0, rbuf1, rbuf2, rbuf3)
        sems = (sem0, sem1, sem2, sem3)

        def idx_copy(c):
            pltpu.make_async_copy(
                x_hbm.at[pl.ds(base + c * chunk, chunk)],
                ibufs[c % 2], semi).start()

        def idx_wait(c):
            pltpu.make_async_copy(
                x_hbm.at[pl.ds(base + c * chunk, chunk)],
                ibufs[c % 2], semi).wait()

        def fire(ib, i, rbi):
            for j in range(2):
                pltpu.make_async_copy(
                    emb_hbm.at[ibufs[ib].at[i, j]],
                    rbufs[rbi].at[j], sems[rbi]).start()

        def drain(ib, i, rbi):
            for j in range(2):
                pltpu.make_async_copy(
                    emb_hbm.at[ibufs[ib].at[i, j]],
                    rbufs[rbi].at[j], sems[rbi]).wait()

        def reduce(rbi, sl):
            rb = rbufs[rbi]
            zero = jnp.zeros((LANES,), jnp.float32)
            LO = pl.ds(0, LANES)
            HI = pl.ds(LANES, LANES)
            mask_hi = jnp.full((LANES,), -65536, jnp.int32)

            def lo_half(w):
                return plsc.bitcast(w << 16, jnp.float32)

            def hi_half(w):
                return plsc.bitcast(w & mask_hi, jnp.float32)

            def rbody(r4, accs):
                a0, a1, a2, a3, a4, a5, a6, a7 = accs
                r = 4 * r4
                w00 = rb[0, r, :]
                w01 = rb[0, r + 1, :]
                w02 = rb[0, r + 2, :]
                w03 = rb[0, r + 3, :]
                w10 = rb[1, r, :]
                w11 = rb[1, r + 1, :]
                w12 = rb[1, r + 2, :]
                w13 = rb[1, r + 3, :]
                a0 = a0 + (lo_half(w00) + lo_half(w02))
                a1 = a1 + (hi_half(w00) + hi_half(w02))
                a2 = a2 + (lo_half(w10) + lo_half(w12))
                a3 = a3 + (hi_half(w10) + hi_half(w12))
                a4 = a4 + (lo_half(w01) + lo_half(w03))
                a5 = a5 + (hi_half(w01) + hi_half(w03))
                a6 = a6 + (lo_half(w11) + lo_half(w13))
                a7 = a7 + (hi_half(w11) + hi_half(w13))
                return (a0, a1, a2, a3, a4, a5, a6, a7)

            a = lax.fori_loop(0, h2 // 4, rbody, (zero,) * 8)
            lo = (a[0] + a[2]) + (a[4] + a[6])
            hi = (a[1] + a[3]) + (a[5] + a[7])
            pooled[sl, LO] = lo
            pooled[sl, HI] = hi

        idx_copy(0)
        idx_wait(0)
        for c in range(nchunks):
            ib = c % 2
            if c + 1 < nchunks:
                idx_copy(c + 1)
            for u in range(4):
                fire(ib, u, u)

            def pbody(p, _, ib=ib, c=c):
                i0 = 4 * p
                last = chunk - 1
                for u in range(4):
                    drain(ib, i0 + u, u)
                    reduce(u, c * chunk + i0 + u)
                    fire(ib, jnp.minimum(i0 + 4 + u, last), u)
                return 0

            lax.fori_loop(0, chunk // 4, pbody, 0)
            for u in range(4):
                drain(ib, chunk - 1, u)
            if c + 1 < nchunks:
                idx_wait(c + 1)

        pltpu.sync_copy(pooled, out_hbm.at[pl.ds(base, spw)])

    return body(x_r, emb_table)


def _tc_linear_relu(sums, fc_w, fc_b2, inv_h, B, D):
    nblk = 8
    blk = B // nblk

    def body(s_ref, w_ref, b_ref, o_ref):
        pooled = s_ref[...] * inv_h
        acc = lax.dot_general(
            pooled, w_ref[...], (((1,), (1,)), ((), ())),
            preferred_element_type=jnp.float32)
        o_ref[...] = jnp.maximum(acc + b_ref[...], 0.0)

    return pl.pallas_call(
        body,
        out_shape=jax.ShapeDtypeStruct((B, D), jnp.float32),
        grid=(nblk,),
        in_specs=[
            pl.BlockSpec((blk, D), lambda i: (i, 0)),
            pl.BlockSpec((D, D), lambda i: (0, 0)),
            pl.BlockSpec((1, D), lambda i: (0, 0)),
        ],
        out_specs=pl.BlockSpec((blk, D), lambda i: (i, 0)),
    )(sums, fc_w, fc_b2)


def kernel(x, emb_table, fc_w, fc_b):
    B, H = x.shape
    D = emb_table.shape[1]
    assert B % NW == 0 and H % 2 == 0 and H // 2 <= 128 and D == 2 * LANES
    spw = B // NW
    chunk = 128
    assert spw % chunk == 0 and chunk % 2 == 0

    x_r = x.astype(jnp.int32).reshape(B, 2, H // 2)
    V = emb_table.shape[0]
    n_tail = V % 128
    tail = emb_table[V - n_tail:]
    tail_bits = jax.lax.bitcast_convert_type(
        tail.astype(jnp.bfloat16), jnp.uint16).astype(jnp.int32)
    tail_packed = (tail_bits[:, : D // 2]
                   | (tail_bits[:, D // 2:] << 16)).reshape(-1)
    lin_flat = _sc_detile_table(emb_table.T, tail_packed, V, D)
    lin_table = lin_flat.reshape(V, D // 2)
    sums = _sc_sum_pool(x_r, lin_table, B, H, D, spw, chunk)
    return _tc_linear_relu(sums, fc_w, fc_b.reshape(1, D), 1.0 / H, B, D)


# speedup vs baseline: 2.3193x; 1.0500x over previous
import functools

import jax
import jax.numpy as jnp
from jax import lax
from jax.experimental import pallas as pl
from jax.experimental.pallas import tpu as pltpu
from jax.experimental.pallas import tpu_sc as plsc

NC = 2
NS = 16
NW = NC * NS
LANES = 16


def _sc_detile_table(emb_t, tail_flat, V, D):
    D2 = D // 2
    RW = D2 + 1
    TCOLS = V // 128
    per_w = TCOLS // NW
    CH = 4
    nch = per_w // CH
    CW = CH * 128
    vpw = per_w * 128
    E0 = NW * vpw
    extra_full = TCOLS - NW * per_w
    rem = V - TCOLS * 128
    assert nch >= 3 and nch % 2 == 1
    mesh = plsc.VectorSubcoreMesh(
        core_axis_name="c", subcore_axis_name="s",
        num_cores=NC, num_subcores=NS)

    @functools.partial(
        pl.kernel,
        out_type=jax.ShapeDtypeStruct((V * D2,), jnp.int32),
        mesh=mesh,
        compiler_params=pltpu.CompilerParams(needs_layout_passes=False),
        scratch_types=[
            pltpu.VMEM((D, CW), jnp.float32),
            pltpu.VMEM((D, CW), jnp.float32),
            pltpu.VMEM((CW * RW,), jnp.int32),
            pltpu.VMEM((CW * D2,), jnp.int32),
            pltpu.VMEM((CW * D2,), jnp.int32),
            pltpu.SemaphoreType.DMA,
            pltpu.SemaphoreType.DMA,
            pltpu.SemaphoreType.DMA,
            pltpu.SemaphoreType.DMA,
        ],
    )
    def body(t_hbm, tail_hbm, out_hbm, ib0, ib1, pb, ob0, ob1,
             si0, si1, so0, so1):
        wid = lax.axis_index("s") * NC + lax.axis_index("c")
        vb = pl.multiple_of(wid * vpw, 128)
        ibufs = (ib0, ib1)
        obufs = (ob0, ob1)
        sis = (si0, si1)
        sos = (so0, so1)
        iota16 = jnp.arange(LANES, dtype=jnp.int32)
        iota_rw = iota16 * RW

        def fire_in(c, b):
            start = pl.multiple_of(vb + c * CW, 128)
            pltpu.make_async_copy(
                t_hbm.at[:, pl.ds(start, CW)], ibufs[b], sis[b]).start()

        def drain_in(b):
            pltpu.make_async_copy(
                t_hbm.at[:, pl.ds(vb, CW)], ibufs[b], sis[b]).wait()

        def fire_out(c, b):
            pltpu.make_async_copy(
                obufs[b], out_hbm.at[pl.ds((vb + c * CW) * D2, CW * D2)],
                sos[b]).start()

        def drain_out(b):
            pltpu.make_async_copy(
                obufs[b], out_hbm.at[pl.ds(vb * D2, CW * D2)], sos[b]).wait()

        def transpose_block(ib, ob, nrows):
            for d in range(D2):
                @plsc.parallel_loop(0, nrows, step=LANES)
                def _(v0, d=d):
                    a = ib[d, pl.ds(v0, LANES)]
                    bvals = ib[d + D2, pl.ds(v0, LANES)]
                    w = plsc.bitcast(
                        plsc.pack(a, bvals, format=plsc.PackFormat.INTERLEAVED),
                        jnp.int32)
                    plsc.store_scatter(pb, [iota_rw + (v0 * RW + d)], w)
            @plsc.parallel_loop(0, nrows, step=2, unroll=2)
            def _(r):
                ob[pl.ds(r * D2, LANES)] = pb[pl.ds(r * RW, LANES)]
                ob[pl.ds((r + 1) * D2, LANES)] = pb[pl.ds((r + 1) * RW, LANES)]

        def process(c, b, first):
            drain_in(b)
            if not first:
                drain_out(b)
            transpose_block(ibufs[b], obufs[b], CW)
            fire_out(c, b)
            fire_in(jnp.minimum(c + 2, nch - 1), b)

        fire_in(0, 0)
        fire_in(1, 1)
        process(0, 0, True)
        process(1, 1, True)

        def pbody(p, _):
            process(2 * p, 0, False)
            process(2 * p + 1, 1, False)
            return 0

        lax.fori_loop(1, (nch - 1) // 2, pbody, 0)
        process(nch - 1, 0, False)
        drain_in(0)
        drain_in(1)
        drain_out(0)
        drain_out(1)

        @pl.when(wid < extra_full)
        def _():
            v0 = pl.multiple_of(E0 + wid * 128, 128)
            pltpu.sync_copy(t_hbm.at[:, pl.ds(v0, 128)],
                            ib0.at[:, pl.ds(0, 128)])
            transpose_block(ib0, ob0, 128)
            pltpu.sync_copy(ob0.at[pl.ds(0, 128 * D2)],
                            out_hbm.at[pl.ds(v0 * D2, 128 * D2)])

        if rem:
            @pl.when(wid == extra_full)
            def _():
                v0 = E0 + extra_full * 128
                pltpu.sync_copy(tail_hbm, ob0.at[pl.ds(0, rem * D2)])
                pltpu.sync_copy(ob0.at[pl.ds(0, rem * D2)],
                                out_hbm.at[pl.ds(v0 * D2, rem * D2)])

    return body(emb_t, tail_flat)


def _sc_sum_pool(x_r, emb_table, B, H, D, spw, chunk):
    h2 = H // 2
    nchunks = spw // chunk
    mesh = plsc.VectorSubcoreMesh(
        core_axis_name="c", subcore_axis_name="s",
        num_cores=NC, num_subcores=NS)

    @functools.partial(
        pl.kernel,
        out_type=jax.ShapeDtypeStruct((B, D), jnp.float32),
        mesh=mesh,
        compiler_params=pltpu.CompilerParams(
            use_tc_tiling_on_sc=False, needs_layout_passes=False),
        scratch_types=[
            pltpu.VMEM((chunk, 2, h2), jnp.int32),
            pltpu.VMEM((chunk, 2, h2), jnp.int32),
            pltpu.VMEM((2, h2, D // 2), jnp.int32),
            pltpu.VMEM((2, h2, D // 2), jnp.int32),
            pltpu.VMEM((2, h2, D // 2), jnp.int32),
            pltpu.VMEM((2, h2, D // 2), jnp.int32),
            pltpu.VMEM((2, h2, D // 2), jnp.int32),
            pltpu.VMEM((2, h2, D // 2), jnp.int32),
            pltpu.VMEM((2, h2, D // 2), jnp.int32),
            pltpu.VMEM((2, h2, D // 2), jnp.int32),
            pltpu.VMEM((spw, D), jnp.float32),
            pltpu.SemaphoreType.DMA,
            pltpu.SemaphoreType.DMA,
            pltpu.SemaphoreType.DMA,
            pltpu.SemaphoreType.DMA,
            pltpu.SemaphoreType.DMA,
            pltpu.SemaphoreType.DMA,
            pltpu.SemaphoreType.DMA,
            pltpu.SemaphoreType.DMA,
            pltpu.SemaphoreType.DMA,
        ],
    )
    def body(x_hbm, emb_hbm, out_hbm, ibuf0, ibuf1, rbuf0, rbuf1, rbuf2, rbuf3,
             rbuf4, rbuf5, rbuf6, rbuf7, pooled, semi,
             sem0, sem1, sem2, sem3, sem4, sem5, sem6, sem7):
        wid = lax.axis_index("s") * NC + lax.axis_index("c")
        base = wid * spw
        ibufs = (ibuf0, ibuf1)
        rbufs = (rbuf0, rbuf1, rbuf2, rbuf3, rbuf4, rbuf5, rbuf6, rbuf7)
        sems = (sem0, sem1, sem2, sem3, sem4, sem5, sem6, sem7)

        def idx_copy(c):
            pltpu.make_async_copy(
                x_hbm.at[pl.ds(base + c * chunk, chunk)],
                ibufs[c % 2], semi).start()

        def idx_wait(c):
            pltpu.make_async_copy(
                x_hbm.at[pl.ds(base + c * chunk, chunk)],
                ibufs[c % 2], semi).wait()

        def fire(ib, i, rbi):
            for j in range(2):
                pltpu.make_async_copy(
                    emb_hbm.at[ibufs[ib].at[i, j]],
                    rbufs[rbi].at[j], sems[rbi]).start()

        def drain(ib, i, rbi):
            for j in range(2):
                pltpu.make_async_copy(
                    emb_hbm.at[ibufs[ib].at[i, j]],
                    rbufs[rbi].at[j], sems[rbi]).wait()

        def reduce(rbi, sl):
            rb = rbufs[rbi]
            zero = jnp.zeros((LANES,), jnp.float32)
            LO = pl.ds(0, LANES)
            HI = pl.ds(LANES, LANES)
            mask_hi = jnp.full((LANES,), -65536, jnp.int32)

            def lo_half(w):
                return plsc.bitcast(w << 16, jnp.float32)

            def hi_half(w):
                return plsc.bitcast(w & mask_hi, jnp.float32)

            def rbody(r4, accs):
                a0, a1, a2, a3, a4, a5, a6, a7 = accs
                r = 4 * r4
                w00 = rb[0, r, :]
                w01 = rb[0, r + 1, :]
                w02 = rb[0, r + 2, :]
                w03 = rb[0, r + 3, :]
                w10 = rb[1, r, :]
                w11 = rb[1, r + 1, :]
                w12 = rb[1, r + 2, :]
                w13 = rb[1, r + 3, :]
                a0 = a0 + (lo_half(w00) + lo_half(w02))
                a1 = a1 + (hi_half(w00) + hi_half(w02))
                a2 = a2 + (lo_half(w10) + lo_half(w12))
                a3 = a3 + (hi_half(w10) + hi_half(w12))
                a4 = a4 + (lo_half(w01) + lo_half(w03))
                a5 = a5 + (hi_half(w01) + hi_half(w03))
                a6 = a6 + (lo_half(w11) + lo_half(w13))
                a7 = a7 + (hi_half(w11) + hi_half(w13))
                return (a0, a1, a2, a3, a4, a5, a6, a7)

            a = lax.fori_loop(0, h2 // 4, rbody, (zero,) * 8)
            lo = (a[0] + a[2]) + (a[4] + a[6])
            hi = (a[1] + a[3]) + (a[5] + a[7])
            pooled[sl, LO] = lo
            pooled[sl, HI] = hi

        idx_copy(0)
        idx_wait(0)
        for c in range(nchunks):
            ib = c % 2
            if c + 1 < nchunks:
                idx_copy(c + 1)
            for u in range(8):
                fire(ib, u, u)

            def pbody(p, _, ib=ib, c=c):
                i0 = 8 * p
                last = chunk - 1
                for u in range(8):
                    drain(ib, i0 + u, u)
                    reduce(u, c * chunk + i0 + u)
                    fire(ib, jnp.minimum(i0 + 8 + u, last), u)
                return 0

            lax.fori_loop(0, chunk // 8, pbody, 0)
            for u in range(8):
                drain(ib, chunk - 1, u)
            if c + 1 < nchunks:
                idx_wait(c + 1)

        pltpu.sync_copy(pooled, out_hbm.at[pl.ds(base, spw)])

    return body(x_r, emb_table)


def _tc_linear_relu(sums, fc_w, fc_b2, inv_h, B, D):
    nblk = 8
    blk = B // nblk

    def body(s_ref, w_ref, b_ref, o_ref):
        pooled = s_ref[...] * inv_h
        acc = lax.dot_general(
            pooled, w_ref[...], (((1,), (1,)), ((), ())),
            preferred_element_type=jnp.float32)
        o_ref[...] = jnp.maximum(acc + b_ref[...], 0.0)

    return pl.pallas_call(
        body,
        out_shape=jax.ShapeDtypeStruct((B, D), jnp.float32),
        grid=(nblk,),
        in_specs=[
            pl.BlockSpec((blk, D), lambda i: (i, 0)),
            pl.BlockSpec((D, D), lambda i: (0, 0)),
            pl.BlockSpec((1, D), lambda i: (0, 0)),
        ],
        out_specs=pl.BlockSpec((blk, D), lambda i: (i, 0)),
    )(sums, fc_w, fc_b2)


def kernel(x, emb_table, fc_w, fc_b):
    B, H = x.shape
    D = emb_table.shape[1]
    assert B % NW == 0 and H % 2 == 0 and H // 2 <= 128 and D == 2 * LANES
    spw = B // NW
    chunk = 128
    assert spw % chunk == 0 and chunk % 2 == 0

    x_r = x.astype(jnp.int32).reshape(B, 2, H // 2)
    V = emb_table.shape[0]
    n_tail = V % 128
    tail = emb_table[V - n_tail:]
    tail_bits = jax.lax.bitcast_convert_type(
        tail.astype(jnp.bfloat16), jnp.uint16).astype(jnp.int32)
    tail_packed = (tail_bits[:, : D // 2]
                   | (tail_bits[:, D // 2:] << 16)).reshape(-1)
    lin_flat = _sc_detile_table(emb_table.T, tail_packed, V, D)
    lin_table = lin_flat.reshape(V, D // 2)
    sums = _sc_sum_pool(x_r, lin_table, B, H, D, spw, chunk)
    return _tc_linear_relu(sums, fc_w, fc_b.reshape(1, D), 1.0 / H, B, D)


# speedup vs baseline: 2.4027x; 1.0360x over previous
import functools

import jax
import jax.numpy as jnp
from jax import lax
from jax.experimental import pallas as pl
from jax.experimental.pallas import tpu as pltpu
from jax.experimental.pallas import tpu_sc as plsc

NC = 2
NS = 16
NW = NC * NS
LANES = 16


def _sc_detile_table(emb_t, tail_flat, V, D):
    D2 = D // 2
    RW = D2 + 1
    TCOLS = V // 128
    per_w = TCOLS // NW
    CH = 4
    nch = per_w // CH
    CW = CH * 128
    vpw = per_w * 128
    E0 = NW * vpw
    extra_full = TCOLS - NW * per_w
    rem = V - TCOLS * 128
    assert nch >= 3 and nch % 2 == 1
    mesh = plsc.VectorSubcoreMesh(
        core_axis_name="c", subcore_axis_name="s",
        num_cores=NC, num_subcores=NS)

    @functools.partial(
        pl.kernel,
        out_type=jax.ShapeDtypeStruct((V * D2,), jnp.int32),
        mesh=mesh,
        compiler_params=pltpu.CompilerParams(needs_layout_passes=False),
        scratch_types=[
            pltpu.VMEM((D, CW), jnp.float32),
            pltpu.VMEM((D, CW), jnp.float32),
            pltpu.VMEM((CW * RW,), jnp.int32),
            pltpu.VMEM((CW * D2,), jnp.int32),
            pltpu.VMEM((CW * D2,), jnp.int32),
            pltpu.SemaphoreType.DMA,
            pltpu.SemaphoreType.DMA,
            pltpu.SemaphoreType.DMA,
            pltpu.SemaphoreType.DMA,
        ],
    )
    def body(t_hbm, tail_hbm, out_hbm, ib0, ib1, pb, ob0, ob1,
             si0, si1, so0, so1):
        wid = lax.axis_index("s") * NC + lax.axis_index("c")
        vb = pl.multiple_of(wid * vpw, 128)
        ibufs = (ib0, ib1)
        obufs = (ob0, ob1)
        sis = (si0, si1)
        sos = (so0, so1)
        iota16 = jnp.arange(LANES, dtype=jnp.int32)
        iota_rw = iota16 * RW

        def fire_in(c, b):
            start = pl.multiple_of(vb + c * CW, 128)
            pltpu.make_async_copy(
                t_hbm.at[:, pl.ds(start, CW)], ibufs[b], sis[b]).start()

        def drain_in(b):
            pltpu.make_async_copy(
                t_hbm.at[:, pl.ds(vb, CW)], ibufs[b], sis[b]).wait()

        def fire_out(c, b):
            pltpu.make_async_copy(
                obufs[b], out_hbm.at[pl.ds((vb + c * CW) * D2, CW * D2)],
                sos[b]).start()

        def drain_out(b):
            pltpu.make_async_copy(
                obufs[b], out_hbm.at[pl.ds(vb * D2, CW * D2)], sos[b]).wait()

        def transpose_block(ib, ob, nrows):
            for d in range(D2):
                @plsc.parallel_loop(0, nrows, step=LANES)
                def _(v0, d=d):
                    a = ib[d, pl.ds(v0, LANES)]
                    bvals = ib[d + D2, pl.ds(v0, LANES)]
                    w = plsc.bitcast(
                        plsc.pack(a, bvals, format=plsc.PackFormat.INTERLEAVED),
                        jnp.int32)
                    plsc.store_scatter(pb, [iota_rw + (v0 * RW + d)], w)
            @plsc.parallel_loop(0, nrows, step=2, unroll=2)
            def _(r):
                ob[pl.ds(r * D2, LANES)] = pb[pl.ds(r * RW, LANES)]
                ob[pl.ds((r + 1) * D2, LANES)] = pb[pl.ds((r + 1) * RW, LANES)]

        def process(c, b, first):
            drain_in(b)
            if not first:
                drain_out(b)
            transpose_block(ibufs[b], obufs[b], CW)
            fire_out(c, b)
            fire_in(jnp.minimum(c + 2, nch - 1), b)

        fire_in(0, 0)
        fire_in(1, 1)
        process(0, 0, True)
        process(1, 1, True)

        def pbody(p, _):
            process(2 * p, 0, False)
            process(2 * p + 1, 1, False)
            return 0

        lax.fori_loop(1, (nch - 1) // 2, pbody, 0)
        process(nch - 1, 0, False)
        drain_in(0)
        drain_in(1)
        drain_out(0)
        drain_out(1)

        @pl.when(wid < extra_full)
        def _():
            v0 = pl.multiple_of(E0 + wid * 128, 128)
            pltpu.sync_copy(t_hbm.at[:, pl.ds(v0, 128)],
                            ib0.at[:, pl.ds(0, 128)])
            transpose_block(ib0, ob0, 128)
            pltpu.sync_copy(ob0.at[pl.ds(0, 128 * D2)],
                            out_hbm.at[pl.ds(v0 * D2, 128 * D2)])

        if rem:
            @pl.when(wid == extra_full)
            def _():
                v0 = E0 + extra_full * 128
                pltpu.sync_copy(tail_hbm, ob0.at[pl.ds(0, rem * D2)])
                pltpu.sync_copy(ob0.at[pl.ds(0, rem * D2)],
                                out_hbm.at[pl.ds(v0 * D2, rem * D2)])

    return body(emb_t, tail_flat)


def _sc_sum_pool(x_r, emb_table, B, H, D, spw, chunk):
    h2 = H // 2
    nchunks = spw // chunk
    mesh = plsc.VectorSubcoreMesh(
        core_axis_name="c", subcore_axis_name="s",
        num_cores=NC, num_subcores=NS)

    @functools.partial(
        pl.kernel,
        out_type=jax.ShapeDtypeStruct((B, D), jnp.float32),
        mesh=mesh,
        compiler_params=pltpu.CompilerParams(
            use_tc_tiling_on_sc=False, needs_layout_passes=False),
        scratch_types=[
            pltpu.VMEM((chunk, H), jnp.int32),
            pltpu.VMEM((chunk, H), jnp.int32),
            pltpu.VMEM((H, D // 2), jnp.int32),
            pltpu.VMEM((H, D // 2), jnp.int32),
            pltpu.VMEM((H, D // 2), jnp.int32),
            pltpu.VMEM((H, D // 2), jnp.int32),
            pltpu.VMEM((H, D // 2), jnp.int32),
            pltpu.VMEM((H, D // 2), jnp.int32),
            pltpu.VMEM((H, D // 2), jnp.int32),
            pltpu.VMEM((H, D // 2), jnp.int32),
            pltpu.VMEM((spw, D), jnp.float32),
            pltpu.SemaphoreType.DMA,
            pltpu.SemaphoreType.DMA,
            pltpu.SemaphoreType.DMA,
            pltpu.SemaphoreType.DMA,
            pltpu.SemaphoreType.DMA,
            pltpu.SemaphoreType.DMA,
            pltpu.SemaphoreType.DMA,
            pltpu.SemaphoreType.DMA,
            pltpu.SemaphoreType.DMA,
        ],
    )
    def body(x_hbm, emb_hbm, out_hbm, ibuf0, ibuf1, rbuf0, rbuf1, rbuf2, rbuf3,
             rbuf4, rbuf5, rbuf6, rbuf7, pooled, semi,
             sem0, sem1, sem2, sem3, sem4, sem5, sem6, sem7):
        wid = lax.axis_index("s") * NC + lax.axis_index("c")
        base = wid * spw
        ibufs = (ibuf0, ibuf1)
        rbufs = (rbuf0, rbuf1, rbuf2, rbuf3, rbuf4, rbuf5, rbuf6, rbuf7)
        sems = (sem0, sem1, sem2, sem3, sem4, sem5, sem6, sem7)

        def idx_copy(c):
            pltpu.make_async_copy(
                x_hbm.at[pl.ds(base + c * chunk, chunk)],
                ibufs[c % 2], semi).start()

        def idx_wait(c):
            pltpu.make_async_copy(
                x_hbm.at[pl.ds(base + c * chunk, chunk)],
                ibufs[c % 2], semi).wait()

        SPLITS = ((0, 96), (96, H - 96))

        def fire(ib, i, rbi):
            for off, n in SPLITS:
                pltpu.make_async_copy(
                    emb_hbm.at[ibufs[ib].at[i, pl.ds(off, n)]],
                    rbufs[rbi].at[pl.ds(off, n)], sems[rbi]).start()

        def drain(ib, i, rbi):
            for off, n in SPLITS:
                pltpu.make_async_copy(
                    emb_hbm.at[ibufs[ib].at[i, pl.ds(off, n)]],
                    rbufs[rbi].at[pl.ds(off, n)], sems[rbi]).wait()

        def reduce(rbi, sl):
            rb = rbufs[rbi]
            zero = jnp.zeros((LANES,), jnp.float32)
            LO = pl.ds(0, LANES)
            HI = pl.ds(LANES, LANES)
            mask_hi = jnp.full((LANES,), -65536, jnp.int32)

            def lo_half(w):
                return plsc.bitcast(w << 16, jnp.float32)

            def hi_half(w):
                return plsc.bitcast(w & mask_hi, jnp.float32)

            def rbody(r8, accs):
                a0, a1, a2, a3, a4, a5, a6, a7 = accs
                r = 8 * r8
                w0 = rb[r, :]
                w1 = rb[r + 1, :]
                w2 = rb[r + 2, :]
                w3 = rb[r + 3, :]
                w4 = rb[r + 4, :]
                w5 = rb[r + 5, :]
                w6 = rb[r + 6, :]
                w7 = rb[r + 7, :]
                a0 = a0 + (lo_half(w0) + lo_half(w4))
                a1 = a1 + (hi_half(w0) + hi_half(w4))
                a2 = a2 + (lo_half(w1) + lo_half(w5))
                a3 = a3 + (hi_half(w1) + hi_half(w5))
                a4 = a4 + (lo_half(w2) + lo_half(w6))
                a5 = a5 + (hi_half(w2) + hi_half(w6))
                a6 = a6 + (lo_half(w3) + lo_half(w7))
                a7 = a7 + (hi_half(w3) + hi_half(w7))
                return (a0, a1, a2, a3, a4, a5, a6, a7)

            a = lax.fori_loop(0, H // 8, rbody, (zero,) * 8)
            lo = (a[0] + a[2]) + (a[4] + a[6])
            hi = (a[1] + a[3]) + (a[5] + a[7])
            pooled[sl, LO] = lo
            pooled[sl, HI] = hi

        idx_copy(0)
        idx_wait(0)
        for c in range(nchunks):
            ib = c % 2
            if c + 1 < nchunks:
                idx_copy(c + 1)
            for u in range(8):
                fire(ib, u, u)

            def pbody(p, _, ib=ib, c=c):
                i0 = 8 * p
                last = chunk - 1
                for u in range(8):
                    drain(ib, i0 + u, u)
                    reduce(u, c * chunk + i0 + u)
                    fire(ib, jnp.minimum(i0 + 8 + u, last), u)
                return 0

            lax.fori_loop(0, chunk // 8, pbody, 0)
            for u in range(8):
                drain(ib, chunk - 1, u)
            if c + 1 < nchunks:
                idx_wait(c + 1)

        pltpu.sync_copy(pooled, out_hbm.at[pl.ds(base, spw)])

    return body(x_r, emb_table)


def _tc_linear_relu(sums, fc_w, fc_b2, inv_h, B, D):
    nblk = 8
    blk = B // nblk

    def body(s_ref, w_ref, b_ref, o_ref):
        pooled = s_ref[...] * inv_h
        acc = lax.dot_general(
            pooled, w_ref[...], (((1,), (1,)), ((), ())),
            preferred_element_type=jnp.float32)
        o_ref[...] = jnp.maximum(acc + b_ref[...], 0.0)

    return pl.pallas_call(
        body,
        out_shape=jax.ShapeDtypeStruct((B, D), jnp.float32),
        grid=(nblk,),
        in_specs=[
            pl.BlockSpec((blk, D), lambda i: (i, 0)),
            pl.BlockSpec((D, D), lambda i: (0, 0)),
            pl.BlockSpec((1, D), lambda i: (0, 0)),
        ],
        out_specs=pl.BlockSpec((blk, D), lambda i: (i, 0)),
    )(sums, fc_w, fc_b2)


def kernel(x, emb_table, fc_w, fc_b):
    B, H = x.shape
    D = emb_table.shape[1]
    assert B % NW == 0 and H % 2 == 0 and H // 2 <= 128 and D == 2 * LANES
    spw = B // NW
    chunk = 128
    assert spw % chunk == 0 and chunk % 2 == 0

    x_r = x.astype(jnp.int32)
    V = emb_table.shape[0]
    n_tail = V % 128
    tail = emb_table[V - n_tail:]
    tail_bits = jax.lax.bitcast_convert_type(
        tail.astype(jnp.bfloat16), jnp.uint16).astype(jnp.int32)
    tail_packed = (tail_bits[:, : D // 2]
                   | (tail_bits[:, D // 2:] << 16)).reshape(-1)
    lin_flat = _sc_detile_table(emb_table.T, tail_packed, V, D)
    lin_table = lin_flat.reshape(V, D // 2)
    sums = _sc_sum_pool(x_r, lin_table, B, H, D, spw, chunk)
    return _tc_linear_relu(sums, fc_w, fc_b.reshape(1, D), 1.0 / H, B, D)


# speedup vs baseline: 2.9342x; 1.2212x over previous
import functools

import jax
import jax.numpy as jnp
from jax import lax
from jax.experimental import pallas as pl
from jax.experimental.pallas import tpu as pltpu
from jax.experimental.pallas import tpu_sc as plsc

NC = 2
NS = 16
NW = NC * NS
LANES = 16


def _sc_detile_table(emb_t, tail_flat, V, D):
    D2 = D // 2
    RW = D2 + 1
    TCOLS = V // 128
    per_w = TCOLS // NW
    CH = 4
    nch = per_w // CH
    CW = CH * 128
    vpw = per_w * 128
    E0 = NW * vpw
    extra_full = TCOLS - NW * per_w
    rem = V - TCOLS * 128
    assert nch >= 3 and nch % 2 == 1
    mesh = plsc.VectorSubcoreMesh(
        core_axis_name="c", subcore_axis_name="s",
        num_cores=NC, num_subcores=NS)

    @functools.partial(
        pl.kernel,
        out_type=jax.ShapeDtypeStruct((V * D2,), jnp.int32),
        mesh=mesh,
        compiler_params=pltpu.CompilerParams(needs_layout_passes=False),
        scratch_types=[
            pltpu.VMEM((D, CW), jnp.float32),
            pltpu.VMEM((D, CW), jnp.float32),
            pltpu.VMEM((CW * RW,), jnp.int32),
            pltpu.VMEM((CW * D2,), jnp.int32),
            pltpu.VMEM((CW * D2,), jnp.int32),
            pltpu.SemaphoreType.DMA,
            pltpu.SemaphoreType.DMA,
            pltpu.SemaphoreType.DMA,
            pltpu.SemaphoreType.DMA,
        ],
    )
    def body(t_hbm, tail_hbm, out_hbm, ib0, ib1, pb, ob0, ob1,
             si0, si1, so0, so1):
        wid = lax.axis_index("s") * NC + lax.axis_index("c")
        vb = pl.multiple_of(wid * vpw, 128)
        ibufs = (ib0, ib1)
        obufs = (ob0, ob1)
        sis = (si0, si1)
        sos = (so0, so1)
        iota16 = jnp.arange(LANES, dtype=jnp.int32)
        iota_rw = iota16 * RW

        def fire_in(c, b):
            start = pl.multiple_of(vb + c * CW, 128)
            pltpu.make_async_copy(
                t_hbm.at[:, pl.ds(start, CW)], ibufs[b], sis[b]).start()

        def drain_in(b):
            pltpu.make_async_copy(
                t_hbm.at[:, pl.ds(vb, CW)], ibufs[b], sis[b]).wait()

        def fire_out(c, b):
            pltpu.make_async_copy(
                obufs[b], out_hbm.at[pl.ds((vb + c * CW) * D2, CW * D2)],
                sos[b]).start()

        def drain_out(b):
            pltpu.make_async_copy(
                obufs[b], out_hbm.at[pl.ds(vb * D2, CW * D2)], sos[b]).wait()

        def transpose_block(ib, ob, nrows):
            for d in range(D2):
                @plsc.parallel_loop(0, nrows, step=LANES, unroll=4)
                def _(v0, d=d):
                    a = ib[d, pl.ds(v0, LANES)]
                    bvals = ib[d + D2, pl.ds(v0, LANES)]
                    w = plsc.bitcast(
                        plsc.pack(a, bvals, format=plsc.PackFormat.INTERLEAVED),
                        jnp.int32)
                    plsc.store_scatter(pb, [iota_rw + (v0 * RW + d)], w)
            @plsc.parallel_loop(0, nrows, step=2, unroll=4)
            def _(r):
                ob[pl.ds(r * D2, LANES)] = pb[pl.ds(r * RW, LANES)]
                ob[pl.ds((r + 1) * D2, LANES)] = pb[pl.ds((r + 1) * RW, LANES)]

        def process(c, b, first):
            drain_in(b)
            if not first:
                drain_out(b)
            transpose_block(ibufs[b], obufs[b], CW)
            fire_out(c, b)
            fire_in(jnp.minimum(c + 2, nch - 1), b)

        fire_in(0, 0)
        fire_in(1, 1)
        process(0, 0, True)
        process(1, 1, True)

        def pbody(p, _):
            process(2 * p, 0, False)
            process(2 * p + 1, 1, False)
            return 0

        lax.fori_loop(1, (nch - 1) // 2, pbody, 0)
        process(nch - 1, 0, False)
        drain_in(0)
        drain_in(1)
        drain_out(0)
        drain_out(1)

        @pl.when(wid < extra_full)
        def _():
            v0 = pl.multiple_of(E0 + wid * 128, 128)
            pltpu.sync_copy(t_hbm.at[:, pl.ds(v0, 128)],
                            ib0.at[:, pl.ds(0, 128)])
            transpose_block(ib0, ob0, 128)
            pltpu.sync_copy(ob0.at[pl.ds(0, 128 * D2)],
                            out_hbm.at[pl.ds(v0 * D2, 128 * D2)])

        if rem:
            @pl.when(wid == extra_full)
            def _():
                v0 = E0 + extra_full * 128
                pltpu.sync_copy(tail_hbm, ob0.at[pl.ds(0, rem * D2)])
                pltpu.sync_copy(ob0.at[pl.ds(0, rem * D2)],
                                out_hbm.at[pl.ds(v0 * D2, rem * D2)])

    return body(emb_t, tail_flat)


def _sc_sum_pool(x_r, emb_table, B, H, D, spw, chunk):
    h2 = H // 2
    nchunks = spw // chunk
    mesh = plsc.VectorSubcoreMesh(
        core_axis_name="c", subcore_axis_name="s",
        num_cores=NC, num_subcores=NS)

    @functools.partial(
        pl.kernel,
        out_type=jax.ShapeDtypeStruct((B, D), jnp.float32),
        mesh=mesh,
        compiler_params=pltpu.CompilerParams(
            use_tc_tiling_on_sc=False, needs_layout_passes=False),
        scratch_types=[
            pltpu.VMEM((chunk, H), jnp.int32),
            pltpu.VMEM((chunk, H), jnp.int32),
            pltpu.VMEM((H, D // 2), jnp.int32),
            pltpu.VMEM((H, D // 2), jnp.int32),
            pltpu.VMEM((H, D // 2), jnp.int32),
            pltpu.VMEM((H, D // 2), jnp.int32),
            pltpu.VMEM((H, D // 2), jnp.int32),
            pltpu.VMEM((H, D // 2), jnp.int32),
            pltpu.VMEM((H, D // 2), jnp.int32),
            pltpu.VMEM((H, D // 2), jnp.int32),
            pltpu.VMEM((spw, D), jnp.float32),
            pltpu.SemaphoreType.DMA,
            pltpu.SemaphoreType.DMA,
            pltpu.SemaphoreType.DMA,
            pltpu.SemaphoreType.DMA,
            pltpu.SemaphoreType.DMA,
            pltpu.SemaphoreType.DMA,
            pltpu.SemaphoreType.DMA,
            pltpu.SemaphoreType.DMA,
            pltpu.SemaphoreType.DMA,
        ],
    )
    def body(x_hbm, emb_hbm, out_hbm, ibuf0, ibuf1, rbuf0, rbuf1, rbuf2, rbuf3,
             rbuf4, rbuf5, rbuf6, rbuf7, pooled, semi,
             sem0, sem1, sem2, sem3, sem4, sem5, sem6, sem7):
        wid = lax.axis_index("s") * NC + lax.axis_index("c")
        base = wid * spw
        ibufs = (ibuf0, ibuf1)
        rbufs = (rbuf0, rbuf1, rbuf2, rbuf3, rbuf4, rbuf5, rbuf6, rbuf7)
        sems = (sem0, sem1, sem2, sem3, sem4, sem5, sem6, sem7)

        def idx_copy(c):
            pltpu.make_async_copy(
                x_hbm.at[pl.ds(base + c * chunk, chunk)],
                ibufs[c % 2], semi).start()

        def idx_wait(c):
            pltpu.make_async_copy(
                x_hbm.at[pl.ds(base + c * chunk, chunk)],
                ibufs[c % 2], semi).wait()

        SPLITS = ((0, 96), (96, H - 96))

        def fire(ib, i, rbi):
            for off, n in SPLITS:
                pltpu.make_async_copy(
                    emb_hbm.at[ibufs[ib].at[i, pl.ds(off, n)]],
                    rbufs[rbi].at[pl.ds(off, n)], sems[rbi]).start()

        def drain(ib, i, rbi):
            for off, n in SPLITS:
                pltpu.make_async_copy(
                    emb_hbm.at[ibufs[ib].at[i, pl.ds(off, n)]],
                    rbufs[rbi].at[pl.ds(off, n)], sems[rbi]).wait()

        def reduce(rbi, sl):
            rb = rbufs[rbi]
            zero = jnp.zeros((LANES,), jnp.float32)
            LO = pl.ds(0, LANES)
            HI = pl.ds(LANES, LANES)
            mask_hi = jnp.full((LANES,), -65536, jnp.int32)

            def lo_half(w):
                return plsc.bitcast(w << 16, jnp.float32)

            def hi_half(w):
                return plsc.bitcast(w & mask_hi, jnp.float32)

            def rbody(r8, accs):
                a0, a1, a2, a3, a4, a5, a6, a7 = accs
                r = 8 * r8
                w0 = rb[r, :]
                w1 = rb[r + 1, :]
                w2 = rb[r + 2, :]
                w3 = rb[r + 3, :]
                w4 = rb[r + 4, :]
                w5 = rb[r + 5, :]
                w6 = rb[r + 6, :]
                w7 = rb[r + 7, :]
                a0 = a0 + (lo_half(w0) + lo_half(w4))
                a1 = a1 + (hi_half(w0) + hi_half(w4))
                a2 = a2 + (lo_half(w1) + lo_half(w5))
                a3 = a3 + (hi_half(w1) + hi_half(w5))
                a4 = a4 + (lo_half(w2) + lo_half(w6))
                a5 = a5 + (hi_half(w2) + hi_half(w6))
                a6 = a6 + (lo_half(w3) + lo_half(w7))
                a7 = a7 + (hi_half(w3) + hi_half(w7))
                return (a0, a1, a2, a3, a4, a5, a6, a7)

            a = lax.fori_loop(0, H // 8, rbody, (zero,) * 8)
            lo = (a[0] + a[2]) + (a[4] + a[6])
            hi = (a[1] + a[3]) + (a[5] + a[7])
            pooled[sl, LO] = lo
            pooled[sl, HI] = hi

        idx_copy(0)
        idx_wait(0)
        for c in range(nchunks):
            ib = c % 2
            if c + 1 < nchunks:
                idx_copy(c + 1)
            for u in range(8):
                fire(ib, u, u)

            def pbody(p, _, ib=ib, c=c):
                i0 = 8 * p
                last = chunk - 1
                for u in range(8):
                    drain(ib, i0 + u, u)
                    reduce(u, c * chunk + i0 + u)
                    fire(ib, jnp.minimum(i0 + 8 + u, last), u)
                return 0

            lax.fori_loop(0, chunk // 8, pbody, 0)
            for u in range(8):
                drain(ib, chunk - 1, u)
            if c + 1 < nchunks:
                idx_wait(c + 1)

        pltpu.sync_copy(pooled, out_hbm.at[pl.ds(base, spw)])

    return body(x_r, emb_table)


def _tc_linear_relu(sums, fc_w, fc_b2, inv_h, B, D):
    nblk = 8
    blk = B // nblk

    def body(s_ref, w_ref, b_ref, o_ref):
        pooled = s_ref[...] * inv_h
        acc = lax.dot_general(
            pooled, w_ref[...], (((1,), (1,)), ((), ())),
            preferred_element_type=jnp.float32)
        o_ref[...] = jnp.maximum(acc + b_ref[...], 0.0)

    return pl.pallas_call(
        body,
        out_shape=jax.ShapeDtypeStruct((B, D), jnp.float32),
        grid=(nblk,),
        in_specs=[
            pl.BlockSpec((blk, D), lambda i: (i, 0)),
            pl.BlockSpec((D, D), lambda i: (0, 0)),
            pl.BlockSpec((1, D), lambda i: (0, 0)),
        ],
        out_specs=pl.BlockSpec((blk, D), lambda i: (i, 0)),
    )(sums, fc_w, fc_b2)


def kernel(x, emb_table, fc_w, fc_b):
    B, H = x.shape
    D = emb_table.shape[1]
    assert B % NW == 0 and H % 2 == 0 and H // 2 <= 128 and D == 2 * LANES
    spw = B // NW
    chunk = 128
    assert spw % chunk == 0 and chunk % 2 == 0

    x_r = x.astype(jnp.int32)
    V = emb_table.shape[0]
    n_tail = V % 128
    tail = emb_table[V - n_tail:]
    tail_bits = jax.lax.bitcast_convert_type(
        tail.astype(jnp.bfloat16), jnp.uint16).astype(jnp.int32)
    tail_packed = (tail_bits[:, : D // 2]
                   | (tail_bits[:, D // 2:] << 16)).reshape(-1)
    lin_flat = _sc_detile_table(emb_table.T, tail_packed, V, D)
    lin_table = lin_flat.reshape(V, D // 2)
    sums = _sc_sum_pool(x_r, lin_table, B, H, D, spw, chunk)
    return _tc_linear_relu(sums, fc_w, fc_b.reshape(1, D), 1.0 / H, B, D)


# speedup vs baseline: 2.9760x; 1.0143x over previous
import functools

import jax
import jax.numpy as jnp
from jax import lax
from jax.experimental import pallas as pl
from jax.experimental.pallas import tpu as pltpu
from jax.experimental.pallas import tpu_sc as plsc

NC = 2
NS = 16
NW = NC * NS
LANES = 16


def _sc_detile_table(emb_t, tail_flat, V, D):
    D2 = D // 2
    RW = D2 + 1
    TCOLS = V // 128
    per_w = TCOLS // NW
    CH = 4
    nch = per_w // CH
    CW = CH * 128
    vpw = per_w * 128
    E0 = NW * vpw
    extra_full = TCOLS - NW * per_w
    rem = V - TCOLS * 128
    assert nch >= 3 and nch % 2 == 1
    mesh = plsc.VectorSubcoreMesh(
        core_axis_name="c", subcore_axis_name="s",
        num_cores=NC, num_subcores=NS)

    @functools.partial(
        pl.kernel,
        out_type=jax.ShapeDtypeStruct((V * D2,), jnp.int32),
        mesh=mesh,
        compiler_params=pltpu.CompilerParams(needs_layout_passes=False),
        scratch_types=[
            pltpu.VMEM((D, CW), jnp.float32),
            pltpu.VMEM((D, CW), jnp.float32),
            pltpu.VMEM((CW * RW,), jnp.int32),
            pltpu.VMEM((CW * D2,), jnp.int32),
            pltpu.VMEM((CW * D2,), jnp.int32),
            pltpu.SemaphoreType.DMA,
            pltpu.SemaphoreType.DMA,
            pltpu.SemaphoreType.DMA,
            pltpu.SemaphoreType.DMA,
        ],
    )
    def body(t_hbm, tail_hbm, out_hbm, ib0, ib1, pb, ob0, ob1,
             si0, si1, so0, so1):
        wid = lax.axis_index("s") * NC + lax.axis_index("c")
        vb = pl.multiple_of(wid * vpw, 128)
        ibufs = (ib0, ib1)
        obufs = (ob0, ob1)
        sis = (si0, si1)
        sos = (so0, so1)
        iota16 = jnp.arange(LANES, dtype=jnp.int32)
        iota_rw = iota16 * RW

        def fire_in(c, b):
            start = pl.multiple_of(vb + c * CW, 128)
            pltpu.make_async_copy(
                t_hbm.at[:, pl.ds(start, CW)], ibufs[b], sis[b]).start()

        def drain_in(b):
            pltpu.make_async_copy(
                t_hbm.at[:, pl.ds(vb, CW)], ibufs[b], sis[b]).wait()

        def fire_out(c, b):
            pltpu.make_async_copy(
                obufs[b], out_hbm.at[pl.ds((vb + c * CW) * D2, CW * D2)],
                sos[b]).start()

        def drain_out(b):
            pltpu.make_async_copy(
                obufs[b], out_hbm.at[pl.ds(vb * D2, CW * D2)], sos[b]).wait()

        def transpose_block(ib, ob, nrows):
            for d in range(D2):
                @plsc.parallel_loop(0, nrows, step=LANES, unroll=8)
                def _(v0, d=d):
                    a = ib[d, pl.ds(v0, LANES)]
                    bvals = ib[d + D2, pl.ds(v0, LANES)]
                    w = plsc.bitcast(
                        plsc.pack(a, bvals, format=plsc.PackFormat.INTERLEAVED),
                        jnp.int32)
                    plsc.store_scatter(pb, [iota_rw + (v0 * RW + d)], w)
            @plsc.parallel_loop(0, nrows, step=2, unroll=8)
            def _(r):
                ob[pl.ds(r * D2, LANES)] = pb[pl.ds(r * RW, LANES)]
                ob[pl.ds((r + 1) * D2, LANES)] = pb[pl.ds((r + 1) * RW, LANES)]

        def process(c, b, first):
            drain_in(b)
            if not first:
                drain_out(b)
            transpose_block(ibufs[b], obufs[b], CW)
            fire_out(c, b)
            fire_in(jnp.minimum(c + 2, nch - 1), b)

        fire_in(0, 0)
        fire_in(1, 1)
        process(0, 0, True)
        process(1, 1, True)

        def pbody(p, _):
            process(2 * p, 0, False)
            process(2 * p + 1, 1, False)
            return 0

        lax.fori_loop(1, (nch - 1) // 2, pbody, 0)
        process(nch - 1, 0, False)
        drain_in(0)
        drain_in(1)
        drain_out(0)
        drain_out(1)

        @pl.when(wid < extra_full)
        def _():
            v0 = pl.multiple_of(E0 + wid * 128, 128)
            pltpu.sync_copy(t_hbm.at[:, pl.ds(v0, 128)],
                            ib0.at[:, pl.ds(0, 128)])
            transpose_block(ib0, ob0, 128)
            pltpu.sync_copy(ob0.at[pl.ds(0, 128 * D2)],
                            out_hbm.at[pl.ds(v0 * D2, 128 * D2)])

        if rem:
            @pl.when(wid == extra_full)
            def _():
                v0 = E0 + extra_full * 128
                pltpu.sync_copy(tail_hbm, ob0.at[pl.ds(0, rem * D2)])
                pltpu.sync_copy(ob0.at[pl.ds(0, rem * D2)],
                                out_hbm.at[pl.ds(v0 * D2, rem * D2)])

    return body(emb_t, tail_flat)


def _sc_sum_pool(x_r, emb_table, B, H, D, spw, chunk):
    h2 = H // 2
    nchunks = spw // chunk
    mesh = plsc.VectorSubcoreMesh(
        core_axis_name="c", subcore_axis_name="s",
        num_cores=NC, num_subcores=NS)

    @functools.partial(
        pl.kernel,
        out_type=jax.ShapeDtypeStruct((B, D), jnp.float32),
        mesh=mesh,
        compiler_params=pltpu.CompilerParams(
            use_tc_tiling_on_sc=False, needs_layout_passes=False),
        scratch_types=[
            pltpu.VMEM((chunk, H), jnp.int32),
            pltpu.VMEM((chunk, H), jnp.int32),
            pltpu.VMEM((H, D // 2), jnp.int32),
            pltpu.VMEM((H, D // 2), jnp.int32),
            pltpu.VMEM((H, D // 2), jnp.int32),
            pltpu.VMEM((H, D // 2), jnp.int32),
            pltpu.VMEM((H, D // 2), jnp.int32),
            pltpu.VMEM((H, D // 2), jnp.int32),
            pltpu.VMEM((H, D // 2), jnp.int32),
            pltpu.VMEM((H, D // 2), jnp.int32),
            pltpu.VMEM((spw, D), jnp.float32),
            pltpu.SemaphoreType.DMA,
            pltpu.SemaphoreType.DMA,
            pltpu.SemaphoreType.DMA,
            pltpu.SemaphoreType.DMA,
            pltpu.SemaphoreType.DMA,
            pltpu.SemaphoreType.DMA,
            pltpu.SemaphoreType.DMA,
            pltpu.SemaphoreType.DMA,
            pltpu.SemaphoreType.DMA,
        ],
    )
    def body(x_hbm, emb_hbm, out_hbm, ibuf0, ibuf1, rbuf0, rbuf1, rbuf2, rbuf3,
             rbuf4, rbuf5, rbuf6, rbuf7, pooled, semi,
             sem0, sem1, sem2, sem3, sem4, sem5, sem6, sem7):
        wid = lax.axis_index("s") * NC + lax.axis_index("c")
        base = wid * spw
        ibufs = (ibuf0, ibuf1)
        rbufs = (rbuf0, rbuf1, rbuf2, rbuf3, rbuf4, rbuf5, rbuf6, rbuf7)
        sems = (sem0, sem1, sem2, sem3, sem4, sem5, sem6, sem7)

        def idx_copy(c):
            pltpu.make_async_copy(
                x_hbm.at[pl.ds(base + c * chunk, chunk)],
                ibufs[c % 2], semi).start()

        def idx_wait(c):
            pltpu.make_async_copy(
                x_hbm.at[pl.ds(base + c * chunk, chunk)],
                ibufs[c % 2], semi).wait()

        SPLITS = ((0, 96), (96, H - 96))

        def fire(ib, i, rbi):
            for off, n in SPLITS:
                pltpu.make_async_copy(
                    emb_hbm.at[ibufs[ib].at[i, pl.ds(off, n)]],
                    rbufs[rbi].at[pl.ds(off, n)], sems[rbi]).start()

        def drain(ib, i, rbi):
            for off, n in SPLITS:
                pltpu.make_async_copy(
                    emb_hbm.at[ibufs[ib].at[i, pl.ds(off, n)]],
                    rbufs[rbi].at[pl.ds(off, n)], sems[rbi]).wait()

        def reduce(rbi, sl):
            rb = rbufs[rbi]
            zero = jnp.zeros((LANES,), jnp.float32)
            LO = pl.ds(0, LANES)
            HI = pl.ds(LANES, LANES)
            mask_hi = jnp.full((LANES,), -65536, jnp.int32)

            def lo_half(w):
                return plsc.bitcast(w << 16, jnp.float32)

            def hi_half(w):
                return plsc.bitcast(w & mask_hi, jnp.float32)

            def rbody(r8, accs):
                a0, a1, a2, a3, a4, a5, a6, a7 = accs
                r = 8 * r8
                w0 = rb[r, :]
                w1 = rb[r + 1, :]
                w2 = rb[r + 2, :]
                w3 = rb[r + 3, :]
                w4 = rb[r + 4, :]
                w5 = rb[r + 5, :]
                w6 = rb[r + 6, :]
                w7 = rb[r + 7, :]
                a0 = a0 + (lo_half(w0) + lo_half(w4))
                a1 = a1 + (hi_half(w0) + hi_half(w4))
                a2 = a2 + (lo_half(w1) + lo_half(w5))
                a3 = a3 + (hi_half(w1) + hi_half(w5))
                a4 = a4 + (lo_half(w2) + lo_half(w6))
                a5 = a5 + (hi_half(w2) + hi_half(w6))
                a6 = a6 + (lo_half(w3) + lo_half(w7))
                a7 = a7 + (hi_half(w3) + hi_half(w7))
                return (a0, a1, a2, a3, a4, a5, a6, a7)

            a = lax.fori_loop(0, H // 8, rbody, (zero,) * 8)
            lo = (a[0] + a[2]) + (a[4] + a[6])
            hi = (a[1] + a[3]) + (a[5] + a[7])
            pooled[sl, LO] = lo
            pooled[sl, HI] = hi

        idx_copy(0)
        idx_wait(0)
        for c in range(nchunks):
            ib = c % 2
            if c + 1 < nchunks:
                idx_copy(c + 1)
            for u in range(8):
                fire(ib, u, u)

            def pbody(p, _, ib=ib, c=c):
                i0 = 8 * p
                last = chunk - 1
                for u in range(8):
                    drain(ib, i0 + u, u)
                    reduce(u, c * chunk + i0 + u)
                    fire(ib, jnp.minimum(i0 + 8 + u, last), u)
                return 0

            lax.fori_loop(0, chunk // 8, pbody, 0)
            for u in range(8):
                drain(ib, chunk - 1, u)
            if c + 1 < nchunks:
                idx_wait(c + 1)

        pltpu.sync_copy(pooled, out_hbm.at[pl.ds(base, spw)])

    return body(x_r, emb_table)


def _tc_linear_relu(sums, fc_w, fc_b2, inv_h, B, D):
    nblk = 8
    blk = B // nblk

    def body(s_ref, w_ref, b_ref, o_ref):
        pooled = s_ref[...] * inv_h
        acc = lax.dot_general(
            pooled, w_ref[...], (((1,), (1,)), ((), ())),
            preferred_element_type=jnp.float32)
        o_ref[...] = jnp.maximum(acc + b_ref[...], 0.0)

    return pl.pallas_call(
        body,
        out_shape=jax.ShapeDtypeStruct((B, D), jnp.float32),
        grid=(nblk,),
        in_specs=[
            pl.BlockSpec((blk, D), lambda i: (i, 0)),
            pl.BlockSpec((D, D), lambda i: (0, 0)),
            pl.BlockSpec((1, D), lambda i: (0, 0)),
        ],
        out_specs=pl.BlockSpec((blk, D), lambda i: (i, 0)),
    )(sums, fc_w, fc_b2)


def kernel(x, emb_table, fc_w, fc_b):
    B, H = x.shape
    D = emb_table.shape[1]
    assert B % NW == 0 and H % 2 == 0 and H // 2 <= 128 and D == 2 * LANES
    spw = B // NW
    chunk = 128
    assert spw % chunk == 0 and chunk % 2 == 0

    x_r = x.astype(jnp.int32)
    V = emb_table.shape[0]
    n_tail = V % 128
    tail = emb_table[V - n_tail:]
    tail_bits = jax.lax.bitcast_convert_type(
        tail.astype(jnp.bfloat16), jnp.uint16).astype(jnp.int32)
    tail_packed = (tail_bits[:, : D // 2]
                   | (tail_bits[:, D // 2:] << 16)).reshape(-1)
    lin_flat = _sc_detile_table(emb_table.T, tail_packed, V, D)
    lin_table = lin_flat.reshape(V, D // 2)
    sums = _sc_sum_pool(x_r, lin_table, B, H, D, spw, chunk)
    return _tc_linear_relu(sums, fc_w, fc_b.reshape(1, D), 1.0 / H, B, D)
